# baseline (speedup 1.0000x reference)
import jax
import jax.numpy as jnp
from jax import lax
from jax.experimental import pallas as pl
from jax.experimental.pallas import tpu as pltpu
from jax.experimental.pallas import tpu_sc as plsc

VOCAB = 100000
EMBED_DIM = 64
BATCH = 4096
HIST = 50

NC = 2
NS = 16
NW = NC * NS

CH = BATCH // NW
NSLOT = 6
TSLOT = 2
LOOKAHEAD = 5
LANES = 16
SKEW = CH + 1


def _transpose_chunk(ebuf_s, tile_t, rows16):
    def per_b(b, carry):
        bb = jnp.full((LANES,), b, jnp.int32)
        for k in range(EMBED_DIM // LANES):
            v = ebuf_s[b, pl.ds(k * LANES, LANES)]
            plsc.store_scatter(tile_t, [rows16[k], bb], v)
        return carry
    lax.fori_loop(0, CH, per_b, 0, unroll=4)


def _lookup_body(idxT_h, emb_h, bias_h, out_e, out_b,
                 idx_v, ebuf, tile, bias_v, gsem, wsem, bsem):
    wid = lax.axis_index("s") * NC + lax.axis_index("c")
    b0 = wid * CH

    pltpu.sync_copy(idxT_h.at[:, pl.ds(b0, CH)], idx_v)

    rows16 = [lax.iota(jnp.int32, LANES) + LANES * k
              for k in range(EMBED_DIM // LANES)]

    for h in range(LOOKAHEAD):
        pltpu.async_copy(emb_h.at[idx_v.at[h]], ebuf.at[h], gsem.at[h])

    def do_chunk(h, t):
        s = lax.rem(h, NSLOT)
        pltpu.make_async_copy(emb_h.at[idx_v.at[h]], ebuf.at[s],
                              gsem.at[s]).wait()
        hn = h + LOOKAHEAD
        sn = lax.rem(hn, NSLOT)

        @pl.when(hn < HIST)
        def _():
            pltpu.async_copy(emb_h.at[idx_v.at[hn]], ebuf.at[sn],
                             gsem.at[sn])
        pltpu.async_copy(bias_h.at[idx_v.at[h]], bias_v.at[h], bsem)

        @pl.when(h >= TSLOT)
        def _():
            for f8 in range(8):
                pltpu.make_async_copy(
                    tile.at[t, pl.ds(f8 * 8, 8), pl.ds(0, CH)],
                    out_e.at[0, f8, 0], wsem.at[t]).wait()

        _transpose_chunk(ebuf.at[s], tile.at[t], rows16)
        for f8 in range(8):
            pltpu.async_copy(tile.at[t, pl.ds(f8 * 8, 8), pl.ds(0, CH)],
                             out_e.at[h, f8, wid], wsem.at[t])

    def step(hh, carry):
        do_chunk(hh * 2, 0)
        do_chunk(hh * 2 + 1, 1)
        return carry

    lax.fori_loop(0, HIST // 2, step, 0)

    for t in range(TSLOT):
        for f8 in range(8):
            pltpu.make_async_copy(tile.at[t, pl.ds(f8 * 8, 8), pl.ds(0, CH)],
                                  out_e.at[0, f8, 0], wsem.at[t]).wait()

    pltpu.make_async_copy(bias_h.at[pl.ds(0, HIST * CH)], bias_v,
                          bsem).wait()
    pltpu.sync_copy(bias_v, out_b.at[:, pl.ds(b0, CH)])


def _lookup_call(idxT, emb, bias):
    mesh = plsc.VectorSubcoreMesh(core_axis_name="c", subcore_axis_name="s",
                                  num_cores=NC, num_subcores=NS)
    f32 = jnp.float32
    run = pl.kernel(
        _lookup_body,
        out_type=(
            jax.ShapeDtypeStruct((HIST, 8, NW, 8, CH), f32),
            jax.ShapeDtypeStruct((HIST, BATCH), f32),
        ),
        mesh=mesh,
        compiler_params=pltpu.CompilerParams(use_tc_tiling_on_sc=False,
                                             needs_layout_passes=False),
        scratch_types=[
            pltpu.VMEM((HIST, CH), jnp.int32),
            pltpu.VMEM((NSLOT, CH, EMBED_DIM), f32),
            pltpu.VMEM((TSLOT, EMBED_DIM, SKEW), f32),
            pltpu.VMEM((HIST, CH), f32),
            pltpu.SemaphoreType.DMA((NSLOT,)),
            pltpu.SemaphoreType.DMA((TSLOT,)),
            pltpu.SemaphoreType.DMA,
        ],
    )
    return run(idxT, emb, bias)


@jax.jit
def _glove_sc(wordsT, ctxT, w_embeddings, w_biases, c_embeddings, c_biases):
    we, wb = _lookup_call(wordsT, w_embeddings, w_biases)
    ce, cb = _lookup_call(ctxT, c_embeddings, c_biases)
    return we, wb, ce, cb


def kernel(words, contexts, w_embeddings, w_biases, c_embeddings, c_biases):
    wordsT = words.astype(jnp.int32).T
    ctxT = contexts.astype(jnp.int32).T
    we5, wb, ce5, cb = _glove_sc(wordsT, ctxT,
                                 w_embeddings, w_biases.reshape(VOCAB),
                                 c_embeddings, c_biases.reshape(VOCAB))

    def chain(x5):
        return jnp.transpose(x5, (2, 4, 0, 1, 3)).reshape(BATCH, HIST,
                                                          EMBED_DIM)

    return (
        chain(we5),
        wb.T.reshape(BATCH, HIST, 1),
        chain(ce5),
        cb.T.reshape(BATCH, HIST, 1),
    )


# speedup vs baseline: 1.0500x; 1.0500x over previous
import jax
import jax.numpy as jnp
from jax import lax
from jax.experimental import pallas as pl
from jax.experimental.pallas import tpu as pltpu
from jax.experimental.pallas import tpu_sc as plsc

VOCAB = 100000
EMBED_DIM = 64
BATCH = 4096
HIST = 50

NC = 2
NS = 16
NW = NC * NS

CH = BATCH // NW
NSLOT = 4
TSLOT = 2
LOOKAHEAD = 3
LANES = 16
SKEW = CH + 1


def _transpose_chunk(ebuf_s, tile_t, rows16):
    def per_b(b, carry):
        bb = jnp.full((LANES,), b, jnp.int32)
        for k in range(EMBED_DIM // LANES):
            v = ebuf_s[b, pl.ds(k * LANES, LANES)]
            plsc.store_scatter(tile_t, [rows16[k], bb], v)
        return carry
    lax.fori_loop(0, CH, per_b, 0, unroll=4)


def _lookup_body(idxT_h, emb_h, bias_h, out_e, out_b,
                 idx_v, ebuf, tile, bias_v, gsem, wsem, bsem):
    wid = lax.axis_index("s") * NC + lax.axis_index("c")
    b0 = wid * CH

    pltpu.sync_copy(idxT_h.at[:, pl.ds(b0, CH)], idx_v)

    rows16 = [lax.iota(jnp.int32, LANES) + LANES * k
              for k in range(EMBED_DIM // LANES)]

    for h in range(LOOKAHEAD):
        pltpu.async_copy(emb_h.at[idx_v.at[h]], ebuf.at[h], gsem.at[h])

    def do_chunk(h, t):
        s = lax.rem(h, NSLOT)
        pltpu.make_async_copy(emb_h.at[idx_v.at[h]], ebuf.at[s],
                              gsem.at[s]).wait()
        hn = h + LOOKAHEAD
        sn = lax.rem(hn, NSLOT)

        @pl.when(hn < HIST)
        def _():
            pltpu.async_copy(emb_h.at[idx_v.at[hn]], ebuf.at[sn],
                             gsem.at[sn])
        pltpu.async_copy(bias_h.at[idx_v.at[h]], bias_v.at[h], bsem)

        @pl.when(h >= TSLOT)
        def _():
            for f8 in range(8):
                pltpu.make_async_copy(
                    tile.at[t, pl.ds(f8 * 8, 8), pl.ds(0, CH)],
                    out_e.at[0, f8, 0], wsem.at[t]).wait()

        _transpose_chunk(ebuf.at[s], tile.at[t], rows16)
        for f8 in range(8):
            pltpu.async_copy(tile.at[t, pl.ds(f8 * 8, 8), pl.ds(0, CH)],
                             out_e.at[h, f8, wid], wsem.at[t])

    def step(hh, carry):
        do_chunk(hh * 2, 0)
        do_chunk(hh * 2 + 1, 1)
        return carry

    lax.fori_loop(0, HIST // 2, step, 0)

    for t in range(TSLOT):
        for f8 in range(8):
            pltpu.make_async_copy(tile.at[t, pl.ds(f8 * 8, 8), pl.ds(0, CH)],
                                  out_e.at[0, f8, 0], wsem.at[t]).wait()

    pltpu.make_async_copy(bias_h.at[pl.ds(0, HIST * CH)], bias_v,
                          bsem).wait()
    pltpu.sync_copy(bias_v, out_b.at[:, pl.ds(b0, CH)])


def _lookup_call(idxT, emb, bias):
    mesh = plsc.VectorSubcoreMesh(core_axis_name="c", subcore_axis_name="s",
                                  num_cores=NC, num_subcores=NS)
    f32 = jnp.float32
    run = pl.kernel(
        _lookup_body,
        out_type=(
            jax.ShapeDtypeStruct((HIST, 8, NW, 8, CH), f32),
            jax.ShapeDtypeStruct((HIST, BATCH), f32),
        ),
        mesh=mesh,
        compiler_params=pltpu.CompilerParams(use_tc_tiling_on_sc=False,
                                             needs_layout_passes=False),
        scratch_types=[
            pltpu.VMEM((HIST, CH), jnp.int32),
            pltpu.VMEM((NSLOT, CH, EMBED_DIM), f32),
            pltpu.VMEM((TSLOT, EMBED_DIM, SKEW), f32),
            pltpu.VMEM((HIST, CH), f32),
            pltpu.SemaphoreType.DMA((NSLOT,)),
            pltpu.SemaphoreType.DMA((TSLOT,)),
            pltpu.SemaphoreType.DMA,
        ],
    )
    return run(idxT, emb, bias)


@jax.jit
def _glove_sc(wordsT, ctxT, w_embeddings, w_biases, c_embeddings, c_biases):
    we, wb = _lookup_call(wordsT, w_embeddings, w_biases)
    ce, cb = _lookup_call(ctxT, c_embeddings, c_biases)
    return we, wb, ce, cb


def kernel(words, contexts, w_embeddings, w_biases, c_embeddings, c_biases):
    wordsT = words.astype(jnp.int32).T
    ctxT = contexts.astype(jnp.int32).T
    we5, wb, ce5, cb = _glove_sc(wordsT, ctxT,
                                 w_embeddings, w_biases.reshape(VOCAB),
                                 c_embeddings, c_biases.reshape(VOCAB))

    def chain(x5):
        return jnp.transpose(x5, (2, 4, 0, 1, 3)).reshape(BATCH, HIST,
                                                          EMBED_DIM)

    return (
        chain(we5),
        wb.T.reshape(BATCH, HIST, 1),
        chain(ce5),
        cb.T.reshape(BATCH, HIST, 1),
    )


# speedup vs baseline: 1.0560x; 1.0057x over previous
import jax
import jax.numpy as jnp
from jax import lax
from jax.experimental import pallas as pl
from jax.experimental.pallas import tpu as pltpu
from jax.experimental.pallas import tpu_sc as plsc

VOCAB = 100000
EMBED_DIM = 64
BATCH = 4096
HIST = 50

NC = 2
NS = 16
NW = NC * NS

CH = BATCH // NW
NSLOT = 4
TSLOT = 2
LOOKAHEAD = 3
LANES = 16
SKEW = CH + 1


def _transpose_chunk(ebuf_s, tile_t, rows16):
    def per_b(b, carry):
        bb = jnp.full((LANES,), b, jnp.int32)
        for k in range(EMBED_DIM // LANES):
            v = ebuf_s[b, pl.ds(k * LANES, LANES)]
            plsc.store_scatter(tile_t, [rows16[k], bb], v)
        return carry
    lax.fori_loop(0, CH, per_b, 0, unroll=8)


def _lookup_body(idxT_h, emb_h, bias_h, out_e, out_b,
                 idx_v, ebuf, tile, bias_v, gsem, wsem, bsem):
    wid = lax.axis_index("s") * NC + lax.axis_index("c")
    b0 = wid * CH

    pltpu.sync_copy(idxT_h.at[:, pl.ds(b0, CH)], idx_v)

    rows16 = [lax.iota(jnp.int32, LANES) + LANES * k
              for k in range(EMBED_DIM // LANES)]

    for h in range(LOOKAHEAD):
        pltpu.async_copy(emb_h.at[idx_v.at[h]], ebuf.at[h], gsem.at[h])

    def do_chunk(h, t):
        s = lax.rem(h, NSLOT)
        pltpu.make_async_copy(emb_h.at[idx_v.at[h]], ebuf.at[s],
                              gsem.at[s]).wait()
        hn = h + LOOKAHEAD
        sn = lax.rem(hn, NSLOT)

        @pl.when(hn < HIST)
        def _():
            pltpu.async_copy(emb_h.at[idx_v.at[hn]], ebuf.at[sn],
                             gsem.at[sn])
        pltpu.async_copy(bias_h.at[idx_v.at[h]], bias_v.at[h], bsem)

        @pl.when(h >= TSLOT)
        def _():
            for f8 in range(8):
                pltpu.make_async_copy(
                    tile.at[t, pl.ds(f8 * 8, 8), pl.ds(0, CH)],
                    out_e.at[0, f8, 0], wsem.at[t]).wait()

        _transpose_chunk(ebuf.at[s], tile.at[t], rows16)
        for f8 in range(8):
            pltpu.async_copy(tile.at[t, pl.ds(f8 * 8, 8), pl.ds(0, CH)],
                             out_e.at[h, f8, wid], wsem.at[t])

    def step(hh, carry):
        do_chunk(hh * 2, 0)
        do_chunk(hh * 2 + 1, 1)
        return carry

    lax.fori_loop(0, HIST // 2, step, 0)

    for t in range(TSLOT):
        for f8 in range(8):
            pltpu.make_async_copy(tile.at[t, pl.ds(f8 * 8, 8), pl.ds(0, CH)],
                                  out_e.at[0, f8, 0], wsem.at[t]).wait()

    pltpu.make_async_copy(bias_h.at[pl.ds(0, HIST * CH)], bias_v,
                          bsem).wait()
    pltpu.sync_copy(bias_v, out_b.at[:, pl.ds(b0, CH)])


def _lookup_call(idxT, emb, bias):
    mesh = plsc.VectorSubcoreMesh(core_axis_name="c", subcore_axis_name="s",
                                  num_cores=NC, num_subcores=NS)
    f32 = jnp.float32
    run = pl.kernel(
        _lookup_body,
        out_type=(
            jax.ShapeDtypeStruct((HIST, 8, NW, 8, CH), f32),
            jax.ShapeDtypeStruct((HIST, BATCH), f32),
        ),
        mesh=mesh,
        compiler_params=pltpu.CompilerParams(use_tc_tiling_on_sc=False,
                                             needs_layout_passes=False),
        scratch_types=[
            pltpu.VMEM((HIST, CH), jnp.int32),
            pltpu.VMEM((NSLOT, CH, EMBED_DIM), f32),
            pltpu.VMEM((TSLOT, EMBED_DIM, SKEW), f32),
            pltpu.VMEM((HIST, CH), f32),
            pltpu.SemaphoreType.DMA((NSLOT,)),
            pltpu.SemaphoreType.DMA((TSLOT,)),
            pltpu.SemaphoreType.DMA,
        ],
    )
    return run(idxT, emb, bias)


@jax.jit
def _glove_sc(wordsT, ctxT, w_embeddings, w_biases, c_embeddings, c_biases):
    we, wb = _lookup_call(wordsT, w_embeddings, w_biases)
    ce, cb = _lookup_call(ctxT, c_embeddings, c_biases)
    return we, wb, ce, cb


def kernel(words, contexts, w_embeddings, w_biases, c_embeddings, c_biases):
    wordsT = words.astype(jnp.int32).T
    ctxT = contexts.astype(jnp.int32).T
    we5, wb, ce5, cb = _glove_sc(wordsT, ctxT,
                                 w_embeddings, w_biases.reshape(VOCAB),
                                 c_embeddings, c_biases.reshape(VOCAB))

    def chain(x5):
        return jnp.transpose(x5, (2, 4, 0, 1, 3)).reshape(BATCH, HIST,
                                                          EMBED_DIM)

    return (
        chain(we5),
        wb.T.reshape(BATCH, HIST, 1),
        chain(ce5),
        cb.T.reshape(BATCH, HIST, 1),
    )


# speedup vs baseline: 1.0615x; 1.0052x over previous
import jax
import jax.numpy as jnp
from jax import lax
from jax.experimental import pallas as pl
from jax.experimental.pallas import tpu as pltpu
from jax.experimental.pallas import tpu_sc as plsc

VOCAB = 100000
EMBED_DIM = 64
BATCH = 4096
HIST = 50

NC = 2
NS = 16
NW = NC * NS

CH = BATCH // NW
NSLOT = 4
TSLOT = 3
LOOKAHEAD = 3
LANES = 16
SKEW = CH + 1


def _transpose_chunk(ebuf_s, tile_t, rows16):
    def per_b(b, carry):
        bb = jnp.full((LANES,), b, jnp.int32)
        for k in range(EMBED_DIM // LANES):
            v = ebuf_s[b, pl.ds(k * LANES, LANES)]
            plsc.store_scatter(tile_t, [rows16[k], bb], v)
        return carry
    lax.fori_loop(0, CH, per_b, 0, unroll=8)


def _lookup_body(idxT_h, emb_h, bias_h, out_e, out_b,
                 idx_v, ebuf, tile, bias_v, gsem, wsem, bsem):
    wid = lax.axis_index("s") * NC + lax.axis_index("c")
    b0 = wid * CH

    pltpu.sync_copy(idxT_h.at[:, pl.ds(b0, CH)], idx_v)

    rows16 = [lax.iota(jnp.int32, LANES) + LANES * k
              for k in range(EMBED_DIM // LANES)]

    for h in range(LOOKAHEAD):
        pltpu.async_copy(emb_h.at[idx_v.at[h]], ebuf.at[h], gsem.at[h])

    def do_chunk(h, t):
        s = lax.rem(h, NSLOT)
        pltpu.make_async_copy(emb_h.at[idx_v.at[h]], ebuf.at[s],
                              gsem.at[s]).wait()
        hn = h + LOOKAHEAD
        sn = lax.rem(hn, NSLOT)

        @pl.when(hn < HIST)
        def _():
            pltpu.async_copy(emb_h.at[idx_v.at[hn]], ebuf.at[sn],
                             gsem.at[sn])
        pltpu.async_copy(bias_h.at[idx_v.at[h]], bias_v.at[h], bsem)

        @pl.when(h >= TSLOT)
        def _():
            for f8 in range(8):
                pltpu.make_async_copy(
                    tile.at[t, pl.ds(f8 * 8, 8), pl.ds(0, CH)],
                    out_e.at[0, f8, 0], wsem.at[t]).wait()

        _transpose_chunk(ebuf.at[s], tile.at[t], rows16)
        for f8 in range(8):
            pltpu.async_copy(tile.at[t, pl.ds(f8 * 8, 8), pl.ds(0, CH)],
                             out_e.at[h, f8, wid], wsem.at[t])

    def step(h, carry):
        do_chunk(h, lax.rem(h, TSLOT))
        return carry

    lax.fori_loop(0, HIST, step, 0)

    for t in range(TSLOT):
        for f8 in range(8):
            pltpu.make_async_copy(tile.at[t, pl.ds(f8 * 8, 8), pl.ds(0, CH)],
                                  out_e.at[0, f8, 0], wsem.at[t]).wait()

    pltpu.make_async_copy(bias_h.at[pl.ds(0, HIST * CH)], bias_v,
                          bsem).wait()
    pltpu.sync_copy(bias_v, out_b.at[:, pl.ds(b0, CH)])


def _lookup_call(idxT, emb, bias):
    mesh = plsc.VectorSubcoreMesh(core_axis_name="c", subcore_axis_name="s",
                                  num_cores=NC, num_subcores=NS)
    f32 = jnp.float32
    run = pl.kernel(
        _lookup_body,
        out_type=(
            jax.ShapeDtypeStruct((HIST, 8, NW, 8, CH), f32),
            jax.ShapeDtypeStruct((HIST, BATCH), f32),
        ),
        mesh=mesh,
        compiler_params=pltpu.CompilerParams(use_tc_tiling_on_sc=False,
                                             needs_layout_passes=False),
        scratch_types=[
            pltpu.VMEM((HIST, CH), jnp.int32),
            pltpu.VMEM((NSLOT, CH, EMBED_DIM), f32),
            pltpu.VMEM((TSLOT, EMBED_DIM, SKEW), f32),
            pltpu.VMEM((HIST, CH), f32),
            pltpu.SemaphoreType.DMA((NSLOT,)),
            pltpu.SemaphoreType.DMA((TSLOT,)),
            pltpu.SemaphoreType.DMA,
        ],
    )
    return run(idxT, emb, bias)


@jax.jit
def _glove_sc(wordsT, ctxT, w_embeddings, w_biases, c_embeddings, c_biases):
    we, wb = _lookup_call(wordsT, w_embeddings, w_biases)
    ce, cb = _lookup_call(ctxT, c_embeddings, c_biases)
    return we, wb, ce, cb


def kernel(words, contexts, w_embeddings, w_biases, c_embeddings, c_biases):
    wordsT = words.astype(jnp.int32).T
    ctxT = contexts.astype(jnp.int32).T
    we5, wb, ce5, cb = _glove_sc(wordsT, ctxT,
                                 w_embeddings, w_biases.reshape(VOCAB),
                                 c_embeddings, c_biases.reshape(VOCAB))

    def chain(x5):
        return jnp.transpose(x5, (2, 4, 0, 1, 3)).reshape(BATCH, HIST,
                                                          EMBED_DIM)

    return (
        chain(we5),
        wb.T.reshape(BATCH, HIST, 1),
        chain(ce5),
        cb.T.reshape(BATCH, HIST, 1),
    )
